# baseline (device time: 22687 ns/iter reference)
import jax
import jax.numpy as jnp
from jax import lax
from jax.experimental import pallas as pl
from jax.experimental.pallas import tpu as pltpu


def kernel(x, dy, gamma):
    del gamma
    m, d = x.shape

    def body(x_ref, dy_ref, out_ref, comm_ref, send_sem, recv_sem):
        my_x = lax.axis_index("x")
        my_y = lax.axis_index("y")
        my_z = lax.axis_index("z")
        nbr = (my_x, 1 - my_y, my_z)

        barrier_sem = pltpu.get_barrier_semaphore()
        pl.semaphore_signal(
            barrier_sem, inc=1, device_id=nbr,
            device_id_type=pl.DeviceIdType.MESH,
        )
        pl.semaphore_wait(barrier_sem, 1)

        xv = x_ref[:, :]
        dyv = dy_ref[:, :]
        mu = jnp.mean(xv, axis=1, keepdims=True)
        var = jnp.mean(xv * xv, axis=1, keepdims=True) - mu * mu
        rstd = lax.rsqrt(var + 1e-5)
        xhat = (xv - mu) * rstd
        comm_ref[0, 0, :] = jnp.sum(dyv * xhat, axis=0)
        comm_ref[0, 1, :] = jnp.sum(dyv, axis=0)

        rdma = pltpu.make_async_remote_copy(
            src_ref=comm_ref.at[0],
            dst_ref=comm_ref.at[1],
            send_sem=send_sem,
            recv_sem=recv_sem,
            device_id=nbr,
            device_id_type=pl.DeviceIdType.MESH,
        )
        rdma.start()
        rdma.wait()

        out_ref[:, :] = comm_ref[0] + comm_ref[1]

    return pl.pallas_call(
        body,
        out_shape=jax.ShapeDtypeStruct((2, d), jnp.float32),
        in_specs=[
            pl.BlockSpec(memory_space=pltpu.VMEM),
            pl.BlockSpec(memory_space=pltpu.VMEM),
        ],
        out_specs=pl.BlockSpec(memory_space=pltpu.VMEM),
        scratch_shapes=[
            pltpu.VMEM((2, 2, d), jnp.float32),
            pltpu.SemaphoreType.DMA,
            pltpu.SemaphoreType.DMA,
        ],
        compiler_params=pltpu.CompilerParams(collective_id=0),
    )(x, dy)


# device time: 21143 ns/iter; 1.0730x vs baseline; 1.0730x over previous
import jax
import jax.numpy as jnp
from jax import lax
from jax.experimental import pallas as pl
from jax.experimental.pallas import tpu as pltpu

_BM = 256


def kernel(x, dy, gamma):
    del gamma
    m, d = x.shape
    nblk = m // _BM

    def body(x_ref, dy_ref, out_ref, acc_ref, comm_ref, send_sem, recv_sem):
        i = pl.program_id(0)
        xv = x_ref[:, :]
        dyv = dy_ref[:, :]
        mu = jnp.mean(xv, axis=1, keepdims=True)
        var = jnp.mean(xv * xv, axis=1, keepdims=True) - mu * mu
        rstd = lax.rsqrt(var + 1e-5)
        xhat = (xv - mu) * rstd
        pdg = jnp.sum(dyv * xhat, axis=0)
        pdb = jnp.sum(dyv, axis=0)

        @pl.when(i == 0)
        def _():
            acc_ref[0, :] = pdg
            acc_ref[1, :] = pdb

        @pl.when(i > 0)
        def _():
            acc_ref[0, :] += pdg
            acc_ref[1, :] += pdb

        @pl.when(i == nblk - 1)
        def _():
            my_x = lax.axis_index("x")
            my_y = lax.axis_index("y")
            my_z = lax.axis_index("z")
            nbr = (my_x, 1 - my_y, my_z)

            barrier_sem = pltpu.get_barrier_semaphore()
            pl.semaphore_signal(
                barrier_sem, inc=1, device_id=nbr,
                device_id_type=pl.DeviceIdType.MESH,
            )
            pl.semaphore_wait(barrier_sem, 1)

            rdma = pltpu.make_async_remote_copy(
                src_ref=acc_ref,
                dst_ref=comm_ref,
                send_sem=send_sem,
                recv_sem=recv_sem,
                device_id=nbr,
                device_id_type=pl.DeviceIdType.MESH,
            )
            rdma.start()
            rdma.wait()

            out_ref[:, :] = acc_ref[:, :] + comm_ref[:, :]

    return pl.pallas_call(
        body,
        grid=(nblk,),
        out_shape=jax.ShapeDtypeStruct((2, d), jnp.float32),
        in_specs=[
            pl.BlockSpec((_BM, d), lambda i: (i, 0)),
            pl.BlockSpec((_BM, d), lambda i: (i, 0)),
        ],
        out_specs=pl.BlockSpec((2, d), lambda i: (0, 0)),
        scratch_shapes=[
            pltpu.VMEM((2, d), jnp.float32),
            pltpu.VMEM((2, d), jnp.float32),
            pltpu.SemaphoreType.DMA,
            pltpu.SemaphoreType.DMA,
        ],
        compiler_params=pltpu.CompilerParams(
            collective_id=0,
            dimension_semantics=("arbitrary",),
        ),
    )(x, dy)


# device time: 19207 ns/iter; 1.1812x vs baseline; 1.1008x over previous
import jax
import jax.numpy as jnp
from jax import lax
from jax.experimental import pallas as pl
from jax.experimental.pallas import tpu as pltpu


def kernel(x, dy, gamma):
    del gamma
    m, d = x.shape
    rows = m // 4

    def body(x_hbm, dy_hbm, out_ref, xbuf, dybuf, acc_ref, recv_ref,
             local_sems, send_sems, recv_sems):
        my_x = lax.axis_index("x")
        my_y = lax.axis_index("y")
        my_z = lax.axis_index("z")
        start = (2 * my_x + my_z) * rows

        cp_x = pltpu.make_async_copy(
            x_hbm.at[pl.ds(start, rows), :], xbuf, local_sems.at[0])
        cp_dy = pltpu.make_async_copy(
            dy_hbm.at[pl.ds(start, rows), :], dybuf, local_sems.at[1])
        cp_x.start()
        cp_dy.start()

        nbrs = [
            (my_x, 1 - my_y, my_z),
            (1 - my_x, my_y, my_z),
            (my_x, my_y, 1 - my_z),
        ]
        barrier_sem = pltpu.get_barrier_semaphore()
        for nbr in nbrs:
            pl.semaphore_signal(
                barrier_sem, inc=1, device_id=nbr,
                device_id_type=pl.DeviceIdType.MESH,
            )
        pl.semaphore_wait(barrier_sem, 3)

        cp_x.wait()
        cp_dy.wait()

        xv = xbuf[:, :]
        dyv = dybuf[:, :]
        mu = jnp.mean(xv, axis=1, keepdims=True)
        var = jnp.mean(xv * xv, axis=1, keepdims=True) - mu * mu
        rstd = lax.rsqrt(var + 1e-5)
        xhat = (xv - mu) * rstd
        acc_ref[0, :] = jnp.sum(dyv * xhat, axis=0)
        acc_ref[1, :] = jnp.sum(dyv, axis=0)

        for k, nbr in enumerate(nbrs):
            rdma = pltpu.make_async_remote_copy(
                src_ref=acc_ref,
                dst_ref=recv_ref.at[k],
                send_sem=send_sems.at[k],
                recv_sem=recv_sems.at[k],
                device_id=nbr,
                device_id_type=pl.DeviceIdType.MESH,
            )
            rdma.start()
            rdma.wait()
            acc_ref[:, :] += recv_ref[k]

        out_ref[:, :] = acc_ref[:, :]

    return pl.pallas_call(
        body,
        out_shape=jax.ShapeDtypeStruct((2, d), jnp.float32),
        in_specs=[
            pl.BlockSpec(memory_space=pl.ANY),
            pl.BlockSpec(memory_space=pl.ANY),
        ],
        out_specs=pl.BlockSpec(memory_space=pltpu.VMEM),
        scratch_shapes=[
            pltpu.VMEM((rows, d), jnp.float32),
            pltpu.VMEM((rows, d), jnp.float32),
            pltpu.VMEM((2, d), jnp.float32),
            pltpu.VMEM((3, 2, d), jnp.float32),
            pltpu.SemaphoreType.DMA((2,)),
            pltpu.SemaphoreType.DMA((3,)),
            pltpu.SemaphoreType.DMA((3,)),
        ],
        compiler_params=pltpu.CompilerParams(collective_id=0),
    )(x, dy)


# device time: 16466 ns/iter; 1.3778x vs baseline; 1.1665x over previous
import jax
import jax.numpy as jnp
from jax import lax
from jax.experimental import pallas as pl
from jax.experimental.pallas import tpu as pltpu

_NDEV = 8


def kernel(x, dy, gamma):
    del gamma
    m, d = x.shape
    rows = m // 4
    half = rows // 2

    def body(x_hbm, dy_hbm, out_ref, xbuf, dybuf, gather_ref,
             local_sems, send_sems, recv_sems):
        my_x = lax.axis_index("x")
        my_y = lax.axis_index("y")
        my_z = lax.axis_index("z")
        my_id = my_x * 4 + my_y * 2 + my_z
        start = (2 * my_x + my_z) * rows

        cps = []
        for c in range(2):
            cpx = pltpu.make_async_copy(
                x_hbm.at[pl.ds(start + c * half, half), :],
                xbuf.at[c], local_sems.at[2 * c])
            cpd = pltpu.make_async_copy(
                dy_hbm.at[pl.ds(start + c * half, half), :],
                dybuf.at[c], local_sems.at[2 * c + 1])
            cpx.start()
            cpd.start()
            cps.append((cpx, cpd))

        barrier_sem = pltpu.get_barrier_semaphore()
        peers = []
        for mask in range(1, _NDEV):
            peer = (my_x ^ (mask >> 2), my_y ^ ((mask >> 1) & 1),
                    my_z ^ (mask & 1))
            peers.append(peer)
            pl.semaphore_signal(
                barrier_sem, inc=1, device_id=peer,
                device_id_type=pl.DeviceIdType.MESH,
            )
        pl.semaphore_wait(barrier_sem, _NDEV - 1)

        pdg = None
        pdb = None
        for c in range(2):
            cps[c][0].wait()
            cps[c][1].wait()
            xv = xbuf[c]
            dyv = dybuf[c]
            mu = jnp.mean(xv, axis=1, keepdims=True)
            var = jnp.mean(xv * xv, axis=1, keepdims=True) - mu * mu
            rstd = lax.rsqrt(var + 1e-5)
            xhat = (xv - mu) * rstd
            g = jnp.sum(dyv * xhat, axis=0)
            b = jnp.sum(dyv, axis=0)
            pdg = g if pdg is None else pdg + g
            pdb = b if pdb is None else pdb + b
        gather_ref[my_id, 0, :] = pdg
        gather_ref[my_id, 1, :] = pdb

        rdmas = []
        for k, peer in enumerate(peers):
            rdma = pltpu.make_async_remote_copy(
                src_ref=gather_ref.at[my_id],
                dst_ref=gather_ref.at[my_id],
                send_sem=send_sems.at[k],
                recv_sem=recv_sems.at[k],
                device_id=peer,
                device_id_type=pl.DeviceIdType.MESH,
            )
            rdma.start()
            rdmas.append(rdma)
        for rdma in rdmas:
            rdma.wait()

        out_ref[:, :] = jnp.sum(gather_ref[:, :, :], axis=0)

    return pl.pallas_call(
        body,
        out_shape=jax.ShapeDtypeStruct((2, d), jnp.float32),
        in_specs=[
            pl.BlockSpec(memory_space=pl.ANY),
            pl.BlockSpec(memory_space=pl.ANY),
        ],
        out_specs=pl.BlockSpec(memory_space=pltpu.VMEM),
        scratch_shapes=[
            pltpu.VMEM((2, half, d), jnp.float32),
            pltpu.VMEM((2, half, d), jnp.float32),
            pltpu.VMEM((_NDEV, 2, d), jnp.float32),
            pltpu.SemaphoreType.DMA((4,)),
            pltpu.SemaphoreType.DMA((_NDEV - 1,)),
            pltpu.SemaphoreType.DMA((_NDEV - 1,)),
        ],
        compiler_params=pltpu.CompilerParams(collective_id=0),
    )(x, dy)
